# baseline (speedup 1.0000x reference)
import functools

import numpy as np
import jax
import jax.numpy as jnp
from jax import lax
from jax.experimental import pallas as pl
from jax.experimental.pallas import tpu as pltpu
from jax.experimental.pallas import tpu_sc as plsc

_N_LEVELS = 16
_F = 2
_HS = 1 << 19
_MASK = _HS - 1
_PRIME = np.int32(np.uint32(2654435761).astype(np.int64) - (1 << 32))

_LOG2_PS = np.log2(1.3819)
_SCALES = []
_RES = []
for _l in range(_N_LEVELS):
    _s = float(np.exp2(_l * _LOG2_PS) * 16 - 1.0)
    _SCALES.append(_s)
    _RES.append(int(np.ceil(_s)) + 1)

_LO = list(range(0, 7))
_MID = list(range(7, 12))
_HI = list(range(12, 16))

_LO_OFF = []
_off = 0
for _l in _LO:
    _LO_OFF.append(_off)
    _off += 2 * _RES[_l] * _RES[_l]
_LO_PACK = ((_off + 127) // 128) * 128

_MID_OFF = []
_off = 0
for _l in _MID:
    _MID_OFF.append(_off)
    _off += _RES[_l] * _RES[_l]
_MID_ROWS = _off

_NC = 2
_NS = 16
_NW = _NC * _NS
_CHUNK = 512
_G = 16
_NG = _CHUNK // _G
_IC = _CHUNK // 128


def _enc_body(x_hbm, lo_hbm, mid_hbm, hi_hbm, enc_hbm,
              lo_v, xr_v, x_v, enc_v, idx_v, w_v, rowsm_v, rowsh_v, sem):
    cid = lax.axis_index("c")
    sid = lax.axis_index("s")
    wid = sid * _NC + cid
    nch = enc_hbm.shape[0] // _NW
    pltpu.sync_copy(lo_hbm, lo_v)
    iota = lax.iota(jnp.int32, _G)
    zero16 = jnp.broadcast_to(jnp.int32(0), (_G,))
    one16 = jnp.broadcast_to(jnp.int32(1), (_G,))

    def chunk_body(ch, _):
        g = wid * nch + ch
        base = g * _CHUNK
        pltpu.sync_copy(x_hbm.at[pl.ds(base, _CHUNK)], xr_v)

        def cvt(j, _c):
            s = pl.ds(j * _G, _G)
            lane = j * _G + iota
            xs = plsc.load_gather(xr_v, [lane, zero16])
            ys = plsc.load_gather(xr_v, [lane, one16])
            x_v[0, s] = (xs + 1.0) * 0.5
            x_v[1, s] = (ys + 1.0) * 0.5
            return _c
        lax.fori_loop(0, _NG, cvt, 0, unroll=2)

        for li in _LO:
            def lo_body(j, _c, scale=_SCALES[li], res=_RES[li],
                        off=_LO_OFF[li], L=li):
                s = pl.ds(j * _G, _G)
                xv = x_v[0, s]
                yv = x_v[1, s]
                px = xv * scale + 0.5
                py = yv * scale + 0.5
                ix = px.astype(jnp.int32)
                iy = py.astype(jnp.int32)
                fx = px - ix.astype(jnp.float32)
                fy = py - iy.astype(jnp.float32)
                cx = jnp.minimum(ix + 1, res - 1)
                cy = jnp.minimum(iy + 1, res - 1)
                r0 = iy * res
                r1 = cy * res
                a00 = (ix + r0) * 2 + off
                a01 = (cx + r0) * 2 + off
                a10 = (ix + r1) * 2 + off
                a11 = (cx + r1) * 2 + off
                wx0 = 1.0 - fx
                wy0 = 1.0 - fy
                w00 = wx0 * wy0
                w01 = fx * wy0
                w10 = wx0 * fy
                w11 = fx * fy
                t00 = plsc.load_gather(lo_v, [a00])
                t01 = plsc.load_gather(lo_v, [a01])
                t10 = plsc.load_gather(lo_v, [a10])
                t11 = plsc.load_gather(lo_v, [a11])
                acc0 = w00 * t00 + w01 * t01 + w10 * t10 + w11 * t11
                u00 = plsc.load_gather(lo_v, [a00 + 1])
                u01 = plsc.load_gather(lo_v, [a01 + 1])
                u10 = plsc.load_gather(lo_v, [a10 + 1])
                u11 = plsc.load_gather(lo_v, [a11 + 1])
                acc1 = w00 * u00 + w01 * u01 + w10 * u10 + w11 * u11
                enc_v[2 * L, s] = acc0
                enc_v[2 * L + 1, s] = acc1
                return _c
            lax.fori_loop(0, _NG, lo_body, 0)

        for k, li in enumerate(_MID):
            def mida(j, _c, scale=_SCALES[li], res=_RES[li], roff=_MID_OFF[k]):
                s = pl.ds(j * _G, _G)
                jc = j >> 3
                sl = pl.ds((j & 7) * _G, _G)
                xv = x_v[0, s]
                yv = x_v[1, s]
                px = xv * scale + 0.5
                py = yv * scale + 0.5
                ix = px.astype(jnp.int32)
                iy = py.astype(jnp.int32)
                fx = px - ix.astype(jnp.float32)
                fy = py - iy.astype(jnp.float32)
                cy = jnp.minimum(iy + 1, res - 1)
                sel = ix >= (res - 1)
                r0 = iy * res + ix + roff
                r1 = cy * res + ix + roff
                wx0 = 1.0 - fx
                wy0 = 1.0 - fy
                w00 = wx0 * wy0
                w01 = fx * wy0
                w10 = wx0 * fy
                w11 = fx * fy
                s01 = jnp.where(sel, w01, 0.0)
                s11 = jnp.where(sel, w11, 0.0)
                idx_v[0, jc, sl] = r0
                idx_v[1, jc, sl] = r1
                w_v[0, s] = w00 + s01
                w_v[1, s] = w01 - s01
                w_v[2, s] = w10 + s11
                w_v[3, s] = w11 - s11
                return _c
            lax.fori_loop(0, _NG, mida, 0)
            cps = []
            for c in range(2):
                for ic in range(_IC):
                    cp = pltpu.make_async_copy(
                        mid_hbm.at[idx_v.at[c, ic]],
                        rowsm_v.at[c * _IC + ic], sem)
                    cp.start()
                    cps.append(cp)
            for cp in cps:
                cp.wait()

            def midb(j, _c, L=li):
                s = pl.ds(j * _G, _G)
                jc = j >> 3
                lane = (j * _G - jc * 128) + iota
                slot_a = jnp.broadcast_to(jc, (_G,))
                slot_b = jnp.broadcast_to(jc + _IC, (_G,))
                w0p = w_v[0, s]
                w1p = w_v[1, s]
                w2p = w_v[2, s]
                w3p = w_v[3, s]
                f0 = jnp.broadcast_to(jnp.int32(0), (_G,))
                f1 = jnp.broadcast_to(jnp.int32(1), (_G,))
                f2 = jnp.broadcast_to(jnp.int32(2), (_G,))
                f3 = jnp.broadcast_to(jnp.int32(3), (_G,))
                a0 = plsc.load_gather(rowsm_v, [slot_a, lane, f0])
                a1 = plsc.load_gather(rowsm_v, [slot_a, lane, f1])
                a2 = plsc.load_gather(rowsm_v, [slot_a, lane, f2])
                a3 = plsc.load_gather(rowsm_v, [slot_a, lane, f3])
                b0 = plsc.load_gather(rowsm_v, [slot_b, lane, f0])
                b1 = plsc.load_gather(rowsm_v, [slot_b, lane, f1])
                b2 = plsc.load_gather(rowsm_v, [slot_b, lane, f2])
                b3 = plsc.load_gather(rowsm_v, [slot_b, lane, f3])
                acc0 = w0p * a0 + w1p * a2 + w2p * b0 + w3p * b2
                acc1 = w0p * a1 + w1p * a3 + w2p * b1 + w3p * b3
                enc_v[2 * L, s] = acc0
                enc_v[2 * L + 1, s] = acc1
                return _c
            lax.fori_loop(0, _NG, midb, 0)

        for k, li in enumerate(_HI):
            def hia(j, _c, scale=_SCALES[li], res=_RES[li], loff=(12 + k) * _HS):
                s = pl.ds(j * _G, _G)
                jc = j >> 3
                sl = pl.ds((j & 7) * _G, _G)
                xv = x_v[0, s]
                yv = x_v[1, s]
                px = xv * scale + 0.5
                py = yv * scale + 0.5
                ix = px.astype(jnp.int32)
                iy = py.astype(jnp.int32)
                fx = px - ix.astype(jnp.float32)
                fy = py - iy.astype(jnp.float32)
                cx = jnp.minimum(ix + 1, res - 1)
                cy = jnp.minimum(iy + 1, res - 1)
                hy0 = iy * _PRIME
                hy1 = cy * _PRIME
                h00 = ((ix ^ hy0) & _MASK) + loff
                h01 = ((cx ^ hy0) & _MASK) + loff
                h10 = ((ix ^ hy1) & _MASK) + loff
                h11 = ((cx ^ hy1) & _MASK) + loff
                wx0 = 1.0 - fx
                wy0 = 1.0 - fy
                idx_v[0, jc, sl] = h00
                idx_v[1, jc, sl] = h01
                idx_v[2, jc, sl] = h10
                idx_v[3, jc, sl] = h11
                w_v[0, s] = wx0 * wy0
                w_v[1, s] = fx * wy0
                w_v[2, s] = wx0 * fy
                w_v[3, s] = fx * fy
                return _c
            lax.fori_loop(0, _NG, hia, 0)
            cps = []
            for c in range(4):
                for ic in range(_IC):
                    cp = pltpu.make_async_copy(
                        hi_hbm.at[idx_v.at[c, ic]],
                        rowsh_v.at[c * _IC + ic], sem)
                    cp.start()
                    cps.append(cp)
            for cp in cps:
                cp.wait()

            def hib(j, _c, L=li):
                s = pl.ds(j * _G, _G)
                jc = j >> 3
                lane = (j * _G - jc * 128) + iota
                f0 = jnp.broadcast_to(jnp.int32(0), (_G,))
                f1 = jnp.broadcast_to(jnp.int32(1), (_G,))
                w0p = w_v[0, s]
                w1p = w_v[1, s]
                w2p = w_v[2, s]
                w3p = w_v[3, s]
                acc0 = jnp.zeros((_G,), jnp.float32)
                acc1 = jnp.zeros((_G,), jnp.float32)
                for c, wv in ((0, w0p), (1, w1p), (2, w2p), (3, w3p)):
                    slot = jnp.broadcast_to(jc + c * _IC, (_G,))
                    t0 = plsc.load_gather(rowsh_v, [slot, lane, f0])
                    t1 = plsc.load_gather(rowsh_v, [slot, lane, f1])
                    acc0 = acc0 + wv * t0
                    acc1 = acc1 + wv * t1
                enc_v[2 * L, s] = acc0
                enc_v[2 * L + 1, s] = acc1
                return _c
            lax.fori_loop(0, _NG, hib, 0)

        pltpu.sync_copy(enc_v, enc_hbm.at[g])
        return _
    lax.fori_loop(0, nch, chunk_body, 0)


def _mlp_body(e_ref, w0, b0, w1, b1, w2, b2, o_ref):
    e = e_ref[0]
    hp = jax.lax.Precision.HIGHEST
    h = jnp.maximum(jax.lax.dot(w0[...], e, precision=hp) + b0[...], 0.0)
    h = jnp.maximum(jax.lax.dot(w1[...], h, precision=hp) + b1[...], 0.0)
    r = jax.lax.dot_general(h, w2[...], (((0,), (0,)), ((), ())),
                            precision=hp) + b2[...]
    o_ref[0] = 1.0 / (1.0 + jnp.exp(-r))


def kernel(x, table, W0, b0, W1, b1, W2, b2):
    n = x.shape[0]
    nchunks = n // _CHUNK

    lo_parts = [table[l, : _RES[l] * _RES[l]].reshape(-1) for l in _LO]
    lo_parts.append(jnp.zeros((_LO_PACK - sum(p.shape[0] for p in lo_parts),),
                              jnp.float32))
    lo_packed = jnp.concatenate(lo_parts)

    mid_parts = []
    for l in _MID:
        t = table[l, : _RES[l] * _RES[l]]
        tn = jnp.concatenate([t[1:], t[-1:]], axis=0)
        mid_parts.append(jnp.concatenate([t, tn], axis=1))
    mid4 = jnp.concatenate(mid_parts, axis=0)

    hi2 = table.reshape(_N_LEVELS * _HS, _F)

    mesh = plsc.VectorSubcoreMesh(core_axis_name="c", subcore_axis_name="s",
                                  num_cores=_NC, num_subcores=_NS)
    enc = pl.kernel(
        _enc_body,
        out_type=jax.ShapeDtypeStruct((nchunks, 2 * _N_LEVELS, _CHUNK),
                                      jnp.float32),
        mesh=mesh,
        compiler_params=pltpu.CompilerParams(needs_layout_passes=False,
                                             use_tc_tiling_on_sc=False),
        scratch_types=[
            pltpu.VMEM((_LO_PACK,), jnp.float32),
            pltpu.VMEM((_CHUNK, 2), jnp.float32),
            pltpu.VMEM((2, _CHUNK), jnp.float32),
            pltpu.VMEM((2 * _N_LEVELS, _CHUNK), jnp.float32),
            pltpu.VMEM((4, _IC, 128), jnp.int32),
            pltpu.VMEM((4, _CHUNK), jnp.float32),
            pltpu.VMEM((2 * _IC, 128, 4), jnp.float32),
            pltpu.VMEM((4 * _IC, 128, 2), jnp.float32),
            pltpu.SemaphoreType.DMA,
        ],
    )(x, lo_packed, mid4, hi2)

    w0t = W0.T
    w1t = W1.T
    b0c = b0[:, None]
    b1c = b1[:, None]
    b2c = b2[None, :]
    hid = W0.shape[1]
    out3 = pl.pallas_call(
        _mlp_body,
        grid=(nchunks,),
        in_specs=[
            pl.BlockSpec((1, 2 * _N_LEVELS, _CHUNK), lambda i: (i, 0, 0)),
            pl.BlockSpec((hid, 2 * _N_LEVELS), lambda i: (0, 0)),
            pl.BlockSpec((hid, 1), lambda i: (0, 0)),
            pl.BlockSpec((hid, hid), lambda i: (0, 0)),
            pl.BlockSpec((hid, 1), lambda i: (0, 0)),
            pl.BlockSpec((hid, 3), lambda i: (0, 0)),
            pl.BlockSpec((1, 3), lambda i: (0, 0)),
        ],
        out_specs=pl.BlockSpec((1, _CHUNK, 3), lambda i: (i, 0, 0)),
        out_shape=jax.ShapeDtypeStruct((nchunks, _CHUNK, 3), jnp.float32),
    )(enc, w0t, b0c, w1t, b1c, W2, b2c)

    return out3.reshape(n, 3)


# speedup vs baseline: 1.1411x; 1.1411x over previous
import functools

import numpy as np
import jax
import jax.numpy as jnp
from jax import lax
from jax.experimental import pallas as pl
from jax.experimental.pallas import tpu as pltpu
from jax.experimental.pallas import tpu_sc as plsc

_N_LEVELS = 16
_F = 2
_HS = 1 << 19
_MASK = _HS - 1
_PRIME = np.int32(np.uint32(2654435761).astype(np.int64) - (1 << 32))

_LOG2_PS = np.log2(1.3819)
_SCALES = []
_RES = []
for _l in range(_N_LEVELS):
    _s = float(np.exp2(_l * _LOG2_PS) * 16 - 1.0)
    _SCALES.append(_s)
    _RES.append(int(np.ceil(_s)) + 1)

_LO = list(range(0, 7))
_MID = list(range(7, 12))
_HI = list(range(12, 16))

_LO_OFF = []
_off = 0
for _l in _LO:
    _LO_OFF.append(_off)
    _off += 2 * _RES[_l] * _RES[_l]
_LO_PACK = ((_off + 127) // 128) * 128

_MID_OFF = []
_off = 0
for _l in _MID:
    _MID_OFF.append(_off)
    _off += _RES[_l] * _RES[_l]
_MID_ROWS = _off

_NC = 2
_NS = 16
_NW = _NC * _NS
_CHUNK = 512
_G = 16
_NG = _CHUNK // _G
_IC = _CHUNK // 128


def _enc_body(x_hbm, lo_hbm, hi_hbm, enc_hbm,
              lo_v, x_v, enc_v, w_v, rowsh_v, sem, *idx_bufs):
    cid = lax.axis_index("c")
    sid = lax.axis_index("s")
    wid = sid * _NC + cid
    nch = enc_hbm.shape[0] // _NW
    pltpu.sync_copy(lo_hbm, lo_v)
    iota = lax.iota(jnp.int32, _G)
    zero16 = jnp.broadcast_to(jnp.int32(0), (_G,))
    one16 = jnp.broadcast_to(jnp.int32(1), (_G,))

    def chunk_body(ch, _):
        g = wid * nch + ch
        base = g * _CHUNK
        pltpu.sync_copy(x_hbm.at[:, pl.ds(base, _CHUNK)], x_v)

        def cvt(j, _c):
            s = pl.ds(j * _G, _G)
            x_v[0, s] = (x_v[0, s] + 1.0) * 0.5
            x_v[1, s] = (x_v[1, s] + 1.0) * 0.5
            return _c
        lax.fori_loop(0, _NG, cvt, 0, unroll=2)

        for li in _LO:
            def lo_body(j, _c, scale=_SCALES[li], res=_RES[li],
                        off=_LO_OFF[li], L=li):
                s = pl.ds(j * _G, _G)
                xv = x_v[0, s]
                yv = x_v[1, s]
                px = xv * scale + 0.5
                py = yv * scale + 0.5
                ix = px.astype(jnp.int32)
                iy = py.astype(jnp.int32)
                fx = px - ix.astype(jnp.float32)
                fy = py - iy.astype(jnp.float32)
                cx = jnp.minimum(ix + 1, res - 1)
                cy = jnp.minimum(iy + 1, res - 1)
                r0 = iy * res
                r1 = cy * res
                a00 = (ix + r0) * 2 + off
                a01 = (cx + r0) * 2 + off
                a10 = (ix + r1) * 2 + off
                a11 = (cx + r1) * 2 + off
                wx0 = 1.0 - fx
                wy0 = 1.0 - fy
                w00 = wx0 * wy0
                w01 = fx * wy0
                w10 = wx0 * fy
                w11 = fx * fy
                t00 = plsc.load_gather(lo_v, [a00])
                t01 = plsc.load_gather(lo_v, [a01])
                t10 = plsc.load_gather(lo_v, [a10])
                t11 = plsc.load_gather(lo_v, [a11])
                acc0 = w00 * t00 + w01 * t01 + w10 * t10 + w11 * t11
                u00 = plsc.load_gather(lo_v, [a00 + 1])
                u01 = plsc.load_gather(lo_v, [a01 + 1])
                u10 = plsc.load_gather(lo_v, [a10 + 1])
                u11 = plsc.load_gather(lo_v, [a11 + 1])
                acc1 = w00 * u00 + w01 * u01 + w10 * u10 + w11 * u11
                enc_v[2 * L, s] = acc0
                enc_v[2 * L + 1, s] = acc1
                return _c
            lax.fori_loop(0, _NG, lo_body, 0)

        for li in _MID + _HI:
            dense = li in _MID

            cps = []
            for ic in range(_IC):
                def hia(jj, _c, ic=ic, scale=_SCALES[li], res=_RES[li],
                        loff=li * _HS, dense=dense):
                    s = pl.ds(ic * 128 + jj * _G, _G)
                    sl = pl.ds(jj * _G, _G)
                    xv = x_v[0, s]
                    yv = x_v[1, s]
                    px = xv * scale + 0.5
                    py = yv * scale + 0.5
                    ix = px.astype(jnp.int32)
                    iy = py.astype(jnp.int32)
                    fx = px - ix.astype(jnp.float32)
                    fy = py - iy.astype(jnp.float32)
                    cx = jnp.minimum(ix + 1, res - 1)
                    cy = jnp.minimum(iy + 1, res - 1)
                    if dense:
                        r0 = iy * res + loff
                        r1 = cy * res + loff
                        h00 = ix + r0
                        h01 = cx + r0
                        h10 = ix + r1
                        h11 = cx + r1
                    else:
                        hy0 = iy * _PRIME
                        hy1 = cy * _PRIME
                        h00 = ((ix ^ hy0) & _MASK) + loff
                        h01 = ((cx ^ hy0) & _MASK) + loff
                        h10 = ((ix ^ hy1) & _MASK) + loff
                        h11 = ((cx ^ hy1) & _MASK) + loff
                    wx0 = 1.0 - fx
                    wy0 = 1.0 - fy
                    idx_bufs[ic * 4 + 0][sl] = h00
                    idx_bufs[ic * 4 + 1][sl] = h01
                    idx_bufs[ic * 4 + 2][sl] = h10
                    idx_bufs[ic * 4 + 3][sl] = h11
                    w_v[0, s] = wx0 * wy0
                    w_v[1, s] = fx * wy0
                    w_v[2, s] = wx0 * fy
                    w_v[3, s] = fx * fy
                    return _c
                lax.fori_loop(0, 8, hia, 0)
                plsc.subcore_barrier()
                for c in range(4):
                    k = ic * 4 + c
                    cp = pltpu.make_async_copy(
                        hi_hbm.at[idx_bufs[k]],
                        rowsh_v.at[k], sem.at[k])
                    cp.start()
                    cps.append(cp)
            for cp in cps:
                cp.wait()

            def hib(j, _c, L=li):
                s = pl.ds(j * _G, _G)
                jc = j >> 3
                lane = (j * _G - jc * 128) + iota
                f0 = jnp.broadcast_to(jnp.int32(0), (_G,))
                f1 = jnp.broadcast_to(jnp.int32(1), (_G,))
                w0p = w_v[0, s]
                w1p = w_v[1, s]
                w2p = w_v[2, s]
                w3p = w_v[3, s]
                acc0 = jnp.zeros((_G,), jnp.float32)
                acc1 = jnp.zeros((_G,), jnp.float32)
                for c, wv in ((0, w0p), (1, w1p), (2, w2p), (3, w3p)):
                    slot = jnp.broadcast_to(jc * 4 + c, (_G,))
                    t0 = plsc.load_gather(rowsh_v, [slot, lane, f0])
                    t1 = plsc.load_gather(rowsh_v, [slot, lane, f1])
                    acc0 = acc0 + wv * t0
                    acc1 = acc1 + wv * t1
                enc_v[2 * L, s] = acc0
                enc_v[2 * L + 1, s] = acc1
                return _c
            lax.fori_loop(0, _NG, hib, 0)

        pltpu.sync_copy(enc_v, enc_hbm.at[g])
        return _
    lax.fori_loop(0, nch, chunk_body, 0)


def _mlp_body(e_ref, w0, b0, w1, b1, w2, b2, o_ref):
    e = e_ref[0]
    hp = jax.lax.Precision.HIGHEST
    h = jnp.maximum(jax.lax.dot(w0[...], e, precision=hp) + b0[...], 0.0)
    h = jnp.maximum(jax.lax.dot(w1[...], h, precision=hp) + b1[...], 0.0)
    r = jax.lax.dot(w2[...], h, precision=hp) + b2[...]
    o_ref[0] = 1.0 / (1.0 + jnp.exp(-r))


def kernel(x, table, W0, b0, W1, b1, W2, b2):
    n = x.shape[0]
    nchunks = n // _CHUNK

    lo_parts = [table[l, : _RES[l] * _RES[l]].reshape(-1) for l in _LO]
    lo_parts.append(jnp.zeros((_LO_PACK - sum(p.shape[0] for p in lo_parts),),
                              jnp.float32))
    lo_packed = jnp.concatenate(lo_parts)

    hi2 = table.reshape(_N_LEVELS * _HS, _F)
    xt = x.T

    mesh = plsc.VectorSubcoreMesh(core_axis_name="c", subcore_axis_name="s",
                                  num_cores=_NC, num_subcores=_NS)
    enc = pl.kernel(
        _enc_body,
        out_type=jax.ShapeDtypeStruct((nchunks, 2 * _N_LEVELS, _CHUNK),
                                      jnp.float32),
        mesh=mesh,
        compiler_params=pltpu.CompilerParams(needs_layout_passes=False,
                                             use_tc_tiling_on_sc=False),
        scratch_types=[
            pltpu.VMEM((_LO_PACK,), jnp.float32),
            pltpu.VMEM((2, _CHUNK), jnp.float32),
            pltpu.VMEM((2 * _N_LEVELS, _CHUNK), jnp.float32),
            pltpu.VMEM((4, _CHUNK), jnp.float32),
            pltpu.VMEM((4 * _IC, 128, 2), jnp.float32),
            pltpu.SemaphoreType.DMA((4 * _IC,)),
        ] + [pltpu.VMEM((128,), jnp.int32) for _ in range(4 * _IC)],
    )(xt, lo_packed, hi2)

    w0t = W0.T
    w1t = W1.T
    w2t = W2.T
    b0c = b0[:, None]
    b1c = b1[:, None]
    b2c = b2[:, None]
    hid = W0.shape[1]
    out3 = pl.pallas_call(
        _mlp_body,
        grid=(nchunks,),
        in_specs=[
            pl.BlockSpec((1, 2 * _N_LEVELS, _CHUNK), lambda i: (i, 0, 0)),
            pl.BlockSpec((hid, 2 * _N_LEVELS), lambda i: (0, 0)),
            pl.BlockSpec((hid, 1), lambda i: (0, 0)),
            pl.BlockSpec((hid, hid), lambda i: (0, 0)),
            pl.BlockSpec((hid, 1), lambda i: (0, 0)),
            pl.BlockSpec((3, hid), lambda i: (0, 0)),
            pl.BlockSpec((3, 1), lambda i: (0, 0)),
        ],
        out_specs=pl.BlockSpec((1, 3, _CHUNK), lambda i: (i, 0, 0)),
        out_shape=jax.ShapeDtypeStruct((nchunks, 3, _CHUNK), jnp.float32),
    )(enc, w0t, b0c, w1t, b1c, w2t, b2c)

    return out3.transpose(0, 2, 1).reshape(n, 3)
